# Initial kernel scaffold; baseline (speedup 1.0000x reference)
import jax
import jax.numpy as jnp
from jax.experimental import pallas as pl


def kernel(text_input_ids, image_features, text_position_ids, image_position_ids, text_segment_ids, image_segment_ids, word_emb, token_type_emb, img_W, img_b, img_proj_ln_g, img_proj_ln_b, text_ln_g, text_ln_b, img_ln_g, img_ln_b):
    raise NotImplementedError("write your pallas kernel here")


# speedup vs baseline: 4.1797x; 4.1797x over previous
import functools

import jax
import jax.numpy as jnp
from jax import lax
from jax.experimental import pallas as pl
from jax.experimental.pallas import tpu as pltpu
from jax.experimental.pallas import tpu_sc as plsc

_B = 1024
_LT = 200
_LI = 20
_H = 128
_DI = 256
_EPS = 1e-12

_NTOK = _B * _LT
_NC = 2
_NS = 16
_NW = _NC * _NS
_CHUNK = 128
_ROWS_PER_W = _NTOK // _NW
_CH_PER_W = _ROWS_PER_W // _CHUNK


def _sc_gather(table, idx3d):
    mesh = plsc.VectorSubcoreMesh(core_axis_name="c", subcore_axis_name="s")

    @functools.partial(
        pl.kernel,
        mesh=mesh,
        out_type=jax.ShapeDtypeStruct((_NTOK, _H), jnp.float32),
        scratch_types=[
            pltpu.VMEM((_CH_PER_W, _CHUNK), jnp.int32),
            pltpu.VMEM((_CHUNK, _H), jnp.float32),
            pltpu.SemaphoreType.DMA,
        ],
    )
    def gather_kernel(table_hbm, idx_hbm, out_hbm, idx_v, rows_v, gsem):
        wid = lax.axis_index("s") * _NC + lax.axis_index("c")
        pltpu.sync_copy(idx_hbm.at[wid], idx_v)
        base = wid * _ROWS_PER_W

        def body(j, carry):
            pltpu.async_copy(table_hbm.at[idx_v.at[j]], rows_v, gsem).wait()
            pltpu.sync_copy(rows_v, out_hbm.at[pl.ds(base + j * _CHUNK, _CHUNK)])
            return carry

        lax.fori_loop(0, _CH_PER_W, body, 0)

    return gather_kernel(table, idx3d)


def _layer_norm(x, g, b):
    m = jnp.mean(x, axis=-1, keepdims=True)
    v = jnp.mean((x - m) ** 2, axis=-1, keepdims=True)
    return (x - m) / jnp.sqrt(v + _EPS) * g + b


_GB = 8


def _tc_body(te_ref, tsid_ref, imf_ref, isid_ref, tte_ref, w_ref, bias_ref,
             g1_ref, b1_ref, gt_ref, bt_ref, gi_ref, bi_ref, out_ref):
    tte0 = tte_ref[0, :]
    tte1 = tte_ref[1, :]

    seg = tsid_ref[...]
    te = te_ref[...] + jnp.where(seg == 0, tte0, tte1)
    out_ref[:, :_LT, :] = _layer_norm(te, gt_ref[0], bt_ref[0])

    x = imf_ref[...].reshape(_GB * _LI, _DI)
    ie = jnp.dot(x, w_ref[...], preferred_element_type=jnp.float32) + bias_ref[0]
    ie = _layer_norm(ie, g1_ref[0], b1_ref[0]).reshape(_GB, _LI, _H)
    iseg = isid_ref[...]
    ie = ie + jnp.where(iseg == 0, tte0, tte1)
    out_ref[:, _LT:, :] = _layer_norm(ie, gi_ref[0], bi_ref[0])


def _tc_embed(te_raw, tsid, imgf, isid, tte, w, bias, g1, b1, gt, bt, gi, bi):
    grid = (_B // _GB,)
    return pl.pallas_call(
        _tc_body,
        grid=grid,
        in_specs=[
            pl.BlockSpec((_GB, _LT, _H), lambda i: (i, 0, 0)),
            pl.BlockSpec((_GB, _LT, 1), lambda i: (i, 0, 0)),
            pl.BlockSpec((_GB, _LI, _DI), lambda i: (i, 0, 0)),
            pl.BlockSpec((_GB, _LI, 1), lambda i: (i, 0, 0)),
            pl.BlockSpec((2, _H), lambda i: (0, 0)),
            pl.BlockSpec((_DI, _H), lambda i: (0, 0)),
            pl.BlockSpec((1, _H), lambda i: (0, 0)),
            pl.BlockSpec((1, _H), lambda i: (0, 0)),
            pl.BlockSpec((1, _H), lambda i: (0, 0)),
            pl.BlockSpec((1, _H), lambda i: (0, 0)),
            pl.BlockSpec((1, _H), lambda i: (0, 0)),
            pl.BlockSpec((1, _H), lambda i: (0, 0)),
            pl.BlockSpec((1, _H), lambda i: (0, 0)),
        ],
        out_specs=pl.BlockSpec((_GB, _LT + _LI, _H), lambda i: (i, 0, 0)),
        out_shape=jax.ShapeDtypeStruct((_B, _LT + _LI, _H), jnp.float32),
    )(te_raw, tsid, imgf, isid, tte, w, bias, g1, b1, gt, bt, gi, bi)


def kernel(text_input_ids, image_features, text_position_ids, image_position_ids,
           text_segment_ids, image_segment_ids,
           word_emb, token_type_emb, img_W, img_b,
           img_proj_ln_g, img_proj_ln_b, text_ln_g, text_ln_b,
           img_ln_g, img_ln_b):
    del text_position_ids, image_position_ids

    idx3d = text_input_ids.reshape(_NW, _CH_PER_W, _CHUNK)
    te_raw = _sc_gather(word_emb, idx3d)
    te_raw = te_raw.reshape(_B, _LT, _H)

    row = lambda p: p.reshape(1, _H)
    return _tc_embed(
        te_raw, text_segment_ids.reshape(_B, _LT, 1),
        image_features, image_segment_ids.reshape(_B, _LI, 1),
        token_type_emb, img_W, row(img_b),
        row(img_proj_ln_g), row(img_proj_ln_b),
        row(text_ln_g), row(text_ln_b),
        row(img_ln_g), row(img_ln_b),
    )


# speedup vs baseline: 6.7232x; 1.6086x over previous
import functools

import jax
import jax.numpy as jnp
from jax import lax
from jax.experimental import pallas as pl
from jax.experimental.pallas import tpu as pltpu
from jax.experimental.pallas import tpu_sc as plsc

_B = 1024
_LT = 200
_LI = 20
_H = 128
_DI = 256
_EPS = 1e-12

_NTOK = _B * _LT
_NC = 2
_NS = 16
_NW = _NC * _NS
_CHUNK = 128
_ROWS_PER_W = _NTOK // _NW
_CH_PER_W = _ROWS_PER_W // _CHUNK
_NBUF = 5
_NGRP = _CH_PER_W // _NBUF


def _sc_gather(table, idx3d):
    mesh = plsc.VectorSubcoreMesh(core_axis_name="c", subcore_axis_name="s")

    @functools.partial(
        pl.kernel,
        mesh=mesh,
        out_type=jax.ShapeDtypeStruct((_NTOK, _H), jnp.float32),
        scratch_types=[
            pltpu.VMEM((_CH_PER_W, _CHUNK), jnp.int32),
            pltpu.VMEM((_NBUF, _CHUNK, _H), jnp.float32),
        ]
        + [pltpu.SemaphoreType.DMA] * (2 * _NBUF),
    )
    def gather_kernel(table_hbm, idx_hbm, out_hbm, idx_v, rows_v, *sems):
        gsem = sems[:_NBUF]
        ssem = sems[_NBUF:]
        wid = lax.axis_index("s") * _NC + lax.axis_index("c")
        pltpu.sync_copy(idx_hbm.at[wid], idx_v)
        base = wid * _ROWS_PER_W

        def fire_gather(j, b):
            pltpu.async_copy(table_hbm.at[idx_v.at[j]], rows_v.at[b], gsem[b])

        for b in range(_NBUF):
            fire_gather(b, b)

        def grp(g, carry):
            for b in range(_NBUF):
                j = g * _NBUF + b
                pltpu.make_async_copy(
                    table_hbm.at[idx_v.at[0]], rows_v.at[b], gsem[b]
                ).wait()
                pltpu.async_copy(
                    rows_v.at[b],
                    out_hbm.at[pl.ds(base + j * _CHUNK, _CHUNK)],
                    ssem[b],
                )
                bk = (b + 3) % _NBUF
                k = j + 3

                @pl.when(k >= _NBUF)
                def _():
                    pltpu.make_async_copy(
                        rows_v.at[bk],
                        out_hbm.at[pl.ds(base, _CHUNK)],
                        ssem[bk],
                    ).wait()

                @pl.when((k >= _NBUF) & (k < _CH_PER_W))
                def _():
                    fire_gather(k, bk)

            return carry

        lax.fori_loop(0, _NGRP, grp, 0)

        for b in ((_CH_PER_W - 2) % _NBUF, (_CH_PER_W - 1) % _NBUF):
            pltpu.make_async_copy(
                rows_v.at[b], out_hbm.at[pl.ds(base, _CHUNK)], ssem[b]
            ).wait()

    return gather_kernel(table, idx3d)


def _layer_norm(x, g, b):
    m = jnp.mean(x, axis=-1, keepdims=True)
    v = jnp.mean((x - m) ** 2, axis=-1, keepdims=True)
    return (x - m) * lax.rsqrt(v + _EPS) * g + b


_BB = 256
_TS = 20
_NSEQ = (_LT + _LI) // _TS


def _tc_body(te_ref, tsid_ref, imf_ref, isid_ref, tte_ref, w_ref, bias_ref,
             g1_ref, b1_ref, gt_ref, bt_ref, gi_ref, bi_ref, out_ref):
    i = pl.program_id(1)
    tte0 = tte_ref[0, :]
    tte1 = tte_ref[1, :]

    @pl.when(i < _NSEQ - 1)
    def _text():
        seg = tsid_ref[...]
        te = te_ref[...] + jnp.where(seg == 0, tte0, tte1)
        out_ref[...] = _layer_norm(te, gt_ref[0], bt_ref[0])

    @pl.when(i == _NSEQ - 1)
    def _image():
        x = imf_ref[...].reshape(_LI * _BB, _DI)
        ie = jnp.dot(x, w_ref[...], preferred_element_type=jnp.float32)
        ie = _layer_norm(ie + bias_ref[0], g1_ref[0], b1_ref[0])
        ie = ie.reshape(_LI, _BB, _H)
        iseg = isid_ref[...]
        ie = ie + jnp.where(iseg == 0, tte0, tte1)
        out_ref[...] = _layer_norm(ie, gi_ref[0], bi_ref[0])


def _tc_embed(te_t, tsid_t, imgf_t, isid_t, tte, w, bias, g1, b1, gt, bt, gi, bi):
    grid = (_B // _BB, _NSEQ)
    last_t = _NSEQ - 2
    return pl.pallas_call(
        _tc_body,
        grid=grid,
        in_specs=[
            pl.BlockSpec((_TS, _BB, _H), lambda j, i: (jnp.minimum(i, last_t), j, 0)),
            pl.BlockSpec((_TS, _BB, 1), lambda j, i: (jnp.minimum(i, last_t), j, 0)),
            pl.BlockSpec((_LI, _BB, _DI), lambda j, i: (0, j, 0)),
            pl.BlockSpec((_LI, _BB, 1), lambda j, i: (0, j, 0)),
            pl.BlockSpec((2, _H), lambda j, i: (0, 0)),
            pl.BlockSpec((_DI, _H), lambda j, i: (0, 0)),
            pl.BlockSpec((1, _H), lambda j, i: (0, 0)),
            pl.BlockSpec((1, _H), lambda j, i: (0, 0)),
            pl.BlockSpec((1, _H), lambda j, i: (0, 0)),
            pl.BlockSpec((1, _H), lambda j, i: (0, 0)),
            pl.BlockSpec((1, _H), lambda j, i: (0, 0)),
            pl.BlockSpec((1, _H), lambda j, i: (0, 0)),
            pl.BlockSpec((1, _H), lambda j, i: (0, 0)),
        ],
        out_specs=pl.BlockSpec((_TS, _BB, _H), lambda j, i: (i, j, 0)),
        out_shape=jax.ShapeDtypeStruct((_LT + _LI, _B, _H), jnp.float32),
    )(te_t, tsid_t, imgf_t, isid_t, tte, w, bias, g1, b1, gt, bt, gi, bi)


def kernel(text_input_ids, image_features, text_position_ids, image_position_ids,
           text_segment_ids, image_segment_ids,
           word_emb, token_type_emb, img_W, img_b,
           img_proj_ln_g, img_proj_ln_b, text_ln_g, text_ln_b,
           img_ln_g, img_ln_b):
    del text_position_ids, image_position_ids

    idx3d = text_input_ids.T.reshape(_NW, _CH_PER_W, _CHUNK)
    te_raw = _sc_gather(word_emb, idx3d)
    te_t = te_raw.reshape(_LT, _B, _H)

    row = lambda p: p.reshape(1, _H)
    out_t = _tc_embed(
        te_t,
        text_segment_ids.T.reshape(_LT, _B, 1),
        image_features.transpose(1, 0, 2),
        image_segment_ids.T.reshape(_LI, _B, 1),
        token_type_emb, img_W, row(img_b),
        row(img_proj_ln_g), row(img_proj_ln_b),
        row(text_ln_g), row(text_ln_b),
        row(img_ln_g), row(img_ln_b),
    )
    return out_t.transpose(1, 0, 2)
